# baseline (speedup 1.0000x reference)
import jax
import jax.numpy as jnp
from jax.experimental import pallas as pl
from jax.experimental.pallas import tpu as pltpu

_K = 16
_B = 32
_G = 4096
_NQ = 16
_CIN = 384
_P = _K * _NQ
_NT = (((1,), (1,)), ((), ()))


def _rowmin(x):
    n = x.shape[1]
    while n > 128:
        n //= 2
        x = jnp.minimum(x[:, :n], x[:, n:])
    return jnp.min(x, axis=1, keepdims=True)


def _topk_half(s, ncand, ref, slot, row_off):
    iota = jax.lax.broadcasted_iota(jnp.int32, (8, ncand), 1)
    for r in range(_K):
        m = _rowmin(s)
        am = _rowmin(jnp.where(s == m, iota, ncand))
        hit = iota == am
        ref[slot, r * _K + row_off:r * _K + row_off + 8, :] = (
            jnp.where(hit, 1.0, 0.0).astype(ref.dtype))
        s = jnp.where(hit, jnp.float32(jnp.inf), s)


def _topk_store(s, ncand, ref, slot):
    _topk_half(s[:8, :], ncand, ref, slot, 0)
    _topk_half(s[8:, :], ncand, ref, slot, 8)


def _rank_sel_store(s, ref, slot):
    ii = jax.lax.broadcasted_iota(jnp.int32, (_NQ, _NQ), 1)
    cnt = jnp.zeros((_NQ, _NQ), jnp.int32)
    for i2 in range(_NQ):
        col = s[:, i2:i2 + 1]
        less = (col < s) | ((col == s) & (i2 < ii))
        cnt = cnt + less.astype(jnp.int32)
    for r in range(_K):
        ref[slot, r * _K:(r + 1) * _K, :] = jnp.where(cnt == r, 1.0, 0.0)


def _blockmax(x):
    z = jnp.maximum(x[:, :-8], x[:, 8:])
    z = jnp.maximum(z[:, :-4], z[:, 4:])
    z = jnp.maximum(z[:, :-2], z[:, 2:])
    z = jnp.maximum(z[:, :-1], z[:, 1:])
    r = jax.lax.broadcasted_iota(jnp.int32, (241, _NQ), 0)
    c = jax.lax.broadcasted_iota(jnp.int32, (241, _NQ), 1)
    epick = jnp.where(r == c * _K, 1.0, 0.0)
    return jnp.dot(z, epick, preferred_element_type=jnp.float32)


def _gn_lrelu(x, gamma, beta, gsize):
    inv_n = 1.0 / (gsize * 256.0)
    parts = []
    for g in range(x.shape[0] // gsize):
        xg = x[g * gsize:(g + 1) * gsize, :]
        mu = jnp.sum(xg) * inv_n
        var = jnp.sum(xg * xg) * inv_n - mu * mu
        y = (xg - mu) * jax.lax.rsqrt(var + 1e-5)
        y = y * gamma[g * gsize:(g + 1) * gsize, :] + beta[g * gsize:(g + 1) * gsize, :]
        parts.append(jnp.where(y >= 0, y, 0.2 * y))
    return jnp.concatenate(parts, axis=0)


def _dgcnn_body(cq_ref, cqt_ref, cgt_ref, fhi_ref, flo_ref, fq_ref, w1_ref,
                g1_ref, b1_ref, w2_ref, g2_ref, b2_ref, out_ref, sel_ref,
                sel2_ref):
    b = pl.program_id(0)
    wslot = jax.lax.rem(b, 2)
    rslot = 1 - wslot


    cq = cq_ref[0]
    cqt = cqt_ref[0]
    cgt = cgt_ref[0]
    cqb = cq.astype(jnp.bfloat16)
    mm = jnp.dot(cqb, cgt.astype(jnp.bfloat16),
                 preferred_element_type=jnp.float32)
    gram = jnp.dot(cqb, cqt.astype(jnp.bfloat16),
                   preferred_element_type=jnp.float32)
    cqsq = jnp.sum(cq * cq, axis=1, keepdims=True)
    cgsq = jnp.sum(cgt * cgt, axis=0, keepdims=True)
    s1 = (-2.0 * mm + cqsq) + cgsq
    nt_row = jnp.sum(cqt * cqt, axis=0, keepdims=True)
    s2 = (-2.0 * gram + cqsq) + nt_row

    pi = jax.lax.broadcasted_iota(jnp.int32, (_P, _NQ), 0)
    qi = jax.lax.broadcasted_iota(jnp.int32, (_P, _NQ), 1)
    eblk = jnp.where(pi // _K == qi, 1.0, 0.0)

    sel = sel_ref[rslot]
    sel2 = sel2_ref[rslot]
    fq = fq_ref[0]
    fgt = (jax.lax.dot_general(sel, fhi_ref[0], _NT,
                               preferred_element_type=jnp.float32)
           + jax.lax.dot_general(sel, flo_ref[0], _NT,
                                 preferred_element_type=jnp.float32))
    fqrep = jax.lax.dot_general(eblk, fq, _NT,
                                preferred_element_type=jnp.float32)
    f1t = jnp.concatenate([fgt - fqrep, fqrep], axis=1).astype(jnp.bfloat16)
    h1 = jax.lax.dot_general(w1_ref[...], f1t, _NT,
                             preferred_element_type=jnp.float32)

    _topk_store(s1, _G, sel_ref, wslot)

    hfull = _gn_lrelu(h1, g1_ref[...], b1_ref[...], 128)
    h = _blockmax(hfull)
    hsel = jax.lax.dot_general(sel2, h, _NT,
                               preferred_element_type=jnp.float32)
    hrep = jax.lax.dot_general(eblk, h, _NT,
                               preferred_element_type=jnp.float32)
    f2t = jnp.concatenate([hsel - hrep, hrep], axis=1).astype(jnp.bfloat16)
    h2 = jax.lax.dot_general(w2_ref[...], f2t, _NT,
                             preferred_element_type=jnp.float32)

    _rank_sel_store(s2, sel2_ref, wslot)

    ofull = _gn_lrelu(h2, g2_ref[...], b2_ref[...], 96)
    out_ref[0] = _blockmax(ofull)


@jax.jit
def kernel(coor, f, coor_q, f_q, W1, g1, b1, W2, g2, b2):
    cgt = jnp.pad(jnp.transpose(coor, (0, 2, 1)), ((0, 0), (0, 5), (0, 0)))
    cq = jnp.pad(coor_q, ((0, 0), (0, 0), (0, 5)))
    cqt = jnp.pad(jnp.transpose(coor_q, (0, 2, 1)), ((0, 0), (0, 5), (0, 0)))
    fhi = f.astype(jnp.bfloat16)
    flo = (f - fhi.astype(jnp.float32)).astype(jnp.bfloat16)
    w1b16 = W1.astype(jnp.bfloat16)
    w2b16 = W2.astype(jnp.bfloat16)
    g1c = g1.reshape(512, 1)
    b1c = b1.reshape(512, 1)
    g2c = g2.reshape(384, 1)
    b2c = b2.reshape(384, 1)

    def topk_idx(*shape):
        return pl.BlockSpec(
            shape, lambda b: (jnp.minimum(b, _B - 1),) + (0,) * (len(shape) - 1))

    def dense_idx(*shape):
        return pl.BlockSpec(
            shape, lambda b: (jnp.maximum(b - 1, 0),) + (0,) * (len(shape) - 1))

    bcast = lambda *shape: pl.BlockSpec(shape, lambda b: (0,) * len(shape))

    return pl.pallas_call(
        _dgcnn_body,
        grid=(_B + 1,),
        in_specs=[
            topk_idx(1, _NQ, 8),
            topk_idx(1, 8, _NQ),
            topk_idx(1, 8, _G),
            dense_idx(1, _CIN, _G),
            dense_idx(1, _CIN, _G),
            dense_idx(1, _CIN, _NQ),
            bcast(512, 768),
            bcast(512, 1),
            bcast(512, 1),
            bcast(384, 1024),
            bcast(384, 1),
            bcast(384, 1),
        ],
        out_specs=dense_idx(1, 384, _NQ),
        out_shape=jax.ShapeDtypeStruct((_B, 384, _NQ), jnp.float32),
        scratch_shapes=[
            pltpu.VMEM((2, _P, _G), jnp.bfloat16),
            pltpu.VMEM((2, _P, _NQ), jnp.float32),
        ],
    )(cq, cqt, cgt, fhi, flo, f_q, w1b16, g1c, b1c, w2b16, g2c, b2c)


# speedup vs baseline: 1.6736x; 1.6736x over previous
import jax
import jax.numpy as jnp
from jax.experimental import pallas as pl
from jax.experimental.pallas import tpu as pltpu

_K = 16
_B = 32
_G = 4096
_NQ = 16
_CIN = 384
_P = _K * _NQ
_NT = (((1,), (1,)), ((), ()))


def _rowmin(x):
    n = x.shape[1]
    while n > 128:
        n //= 2
        x = jnp.minimum(x[:, :n], x[:, n:])
    return jnp.min(x, axis=1, keepdims=True)


def _topk_half(s, ncand, ref, slot, row_off):
    iota = jax.lax.broadcasted_iota(jnp.int32, (8, ncand), 1)
    for r in range(_K):
        m = _rowmin(s)
        am = _rowmin(jnp.where(s == m, iota, ncand))
        hit = iota == am
        ref[slot, r * _K + row_off:r * _K + row_off + 8, :] = (
            jnp.where(hit, 1.0, 0.0).astype(ref.dtype))
        s = jnp.where(hit, jnp.float32(jnp.inf), s)


def _topk_store(s, ncand, ref, slot):
    _topk_half(s[:8, :], ncand, ref, slot, 0)
    _topk_half(s[8:, :], ncand, ref, slot, 8)


def _rank_sel_store(s, ref, slot):
    ii = jax.lax.broadcasted_iota(jnp.int32, (_NQ, _NQ), 1)
    cnt = jnp.zeros((_NQ, _NQ), jnp.int32)
    for i2 in range(_NQ):
        col = s[:, i2:i2 + 1]
        less = (col < s) | ((col == s) & (i2 < ii))
        cnt = cnt + less.astype(jnp.int32)
    for r in range(_K):
        ref[slot, r * _K:(r + 1) * _K, :] = jnp.where(cnt == r, 1.0, 0.0)


def _blockmax(x):
    z = jnp.maximum(x[:, :-8], x[:, 8:])
    z = jnp.maximum(z[:, :-4], z[:, 4:])
    z = jnp.maximum(z[:, :-2], z[:, 2:])
    z = jnp.maximum(z[:, :-1], z[:, 1:])
    r = jax.lax.broadcasted_iota(jnp.int32, (241, _NQ), 0)
    c = jax.lax.broadcasted_iota(jnp.int32, (241, _NQ), 1)
    epick = jnp.where(r == c * _K, 1.0, 0.0)
    return jnp.dot(z, epick, preferred_element_type=jnp.float32)


def _gn_lrelu(x, gamma, beta, gsize):
    inv_n = 1.0 / (gsize * 256.0)
    parts = []
    for g in range(x.shape[0] // gsize):
        xg = x[g * gsize:(g + 1) * gsize, :]
        mu = jnp.sum(xg) * inv_n
        var = jnp.sum(xg * xg) * inv_n - mu * mu
        y = (xg - mu) * jax.lax.rsqrt(var + 1e-5)
        y = y * gamma[g * gsize:(g + 1) * gsize, :] + beta[g * gsize:(g + 1) * gsize, :]
        parts.append(jnp.where(y >= 0, y, 0.2 * y))
    return jnp.concatenate(parts, axis=0)


def _dgcnn_body(cq_ref, cqt_ref, cgt_ref, f_ref, fq_ref, w1_ref,
                g1_ref, b1_ref, w2_ref, g2_ref, b2_ref, out_ref, sel_ref,
                sel2_ref):
    b = pl.program_id(0)
    wslot = jax.lax.rem(b, 2)
    rslot = 1 - wslot


    cq = cq_ref[0]
    cqt = cqt_ref[0]
    cgt = cgt_ref[0]
    cqb = cq.astype(jnp.bfloat16)
    mm = jnp.dot(cqb, cgt.astype(jnp.bfloat16),
                 preferred_element_type=jnp.float32)
    gram = jnp.dot(cqb, cqt.astype(jnp.bfloat16),
                   preferred_element_type=jnp.float32)
    cqsq = jnp.sum(cq * cq, axis=1, keepdims=True)
    cgsq = jnp.sum(cgt * cgt, axis=0, keepdims=True)
    s1 = (-2.0 * mm + cqsq) + cgsq
    nt_row = jnp.sum(cqt * cqt, axis=0, keepdims=True)
    s2 = (-2.0 * gram + cqsq) + nt_row

    pi = jax.lax.broadcasted_iota(jnp.int32, (_P, _NQ), 0)
    qi = jax.lax.broadcasted_iota(jnp.int32, (_P, _NQ), 1)
    eblk = jnp.where(pi // _K == qi, 1.0, 0.0)

    sel = sel_ref[rslot]
    sel2 = sel2_ref[rslot]
    fq = fq_ref[0]
    fgt = jax.lax.dot_general(sel, f_ref[0], _NT,
                              preferred_element_type=jnp.float32)
    fqrep = jax.lax.dot_general(eblk, fq, _NT,
                                preferred_element_type=jnp.float32)
    f1t = jnp.concatenate([fgt - fqrep, fqrep], axis=1).astype(jnp.bfloat16)
    h1 = jax.lax.dot_general(w1_ref[...], f1t, _NT,
                             preferred_element_type=jnp.float32)

    _topk_store(s1, _G, sel_ref, wslot)

    hfull = _gn_lrelu(h1, g1_ref[...], b1_ref[...], 128)
    h = _blockmax(hfull)
    hsel = jax.lax.dot_general(sel2, h, _NT,
                               preferred_element_type=jnp.float32)
    hrep = jax.lax.dot_general(eblk, h, _NT,
                               preferred_element_type=jnp.float32)
    f2t = jnp.concatenate([hsel - hrep, hrep], axis=1).astype(jnp.bfloat16)
    h2 = jax.lax.dot_general(w2_ref[...], f2t, _NT,
                             preferred_element_type=jnp.float32)

    _rank_sel_store(s2, sel2_ref, wslot)

    ofull = _gn_lrelu(h2, g2_ref[...], b2_ref[...], 96)
    out_ref[0] = _blockmax(ofull)


@jax.jit
def kernel(coor, f, coor_q, f_q, W1, g1, b1, W2, g2, b2):
    cgt = jnp.pad(jnp.transpose(coor, (0, 2, 1)), ((0, 0), (0, 5), (0, 0)))
    cq = jnp.pad(coor_q, ((0, 0), (0, 0), (0, 5)))
    cqt = jnp.pad(jnp.transpose(coor_q, (0, 2, 1)), ((0, 0), (0, 5), (0, 0)))
    w1b16 = W1.astype(jnp.bfloat16)
    w2b16 = W2.astype(jnp.bfloat16)
    g1c = g1.reshape(512, 1)
    b1c = b1.reshape(512, 1)
    g2c = g2.reshape(384, 1)
    b2c = b2.reshape(384, 1)

    def topk_idx(*shape):
        return pl.BlockSpec(
            shape, lambda b: (jnp.minimum(b, _B - 1),) + (0,) * (len(shape) - 1))

    def dense_idx(*shape):
        return pl.BlockSpec(
            shape, lambda b: (jnp.maximum(b - 1, 0),) + (0,) * (len(shape) - 1))

    bcast = lambda *shape: pl.BlockSpec(shape, lambda b: (0,) * len(shape))

    return pl.pallas_call(
        _dgcnn_body,
        grid=(_B + 1,),
        in_specs=[
            topk_idx(1, _NQ, 8),
            topk_idx(1, 8, _NQ),
            topk_idx(1, 8, _G),
            dense_idx(1, _CIN, _G),
            dense_idx(1, _CIN, _NQ),
            bcast(512, 768),
            bcast(512, 1),
            bcast(512, 1),
            bcast(384, 1024),
            bcast(384, 1),
            bcast(384, 1),
        ],
        out_specs=dense_idx(1, 384, _NQ),
        out_shape=jax.ShapeDtypeStruct((_B, 384, _NQ), jnp.float32),
        scratch_shapes=[
            pltpu.VMEM((2, _P, _G), jnp.float32),
            pltpu.VMEM((2, _P, _NQ), jnp.float32),
        ],
    )(cq, cqt, cgt, f, f_q, w1b16, g1c, b1c, w2b16, g2c, b2c)


# speedup vs baseline: 1.7099x; 1.0217x over previous
import jax
import jax.numpy as jnp
from jax.experimental import pallas as pl
from jax.experimental.pallas import tpu as pltpu

_K = 16
_B = 32
_G = 4096
_NQ = 16
_CIN = 384
_P = _K * _NQ
_NT = (((1,), (1,)), ((), ()))


def _rowmin(x):
    n = x.shape[1]
    while n > 128:
        n //= 2
        x = jnp.minimum(x[:, :n], x[:, n:])
    return jnp.min(x, axis=1, keepdims=True)


def _topk_iter(s, iota, ncand, ref, slot, r, row_off):
    m = _rowmin(s)
    t = jnp.where(s == m, iota, ncand)
    am = _rowmin(t)
    hit = t == am
    ref[slot, r * _K + row_off:r * _K + row_off + 8, :] = (
        jnp.where(hit, 1.0, 0.0).astype(ref.dtype))
    return jnp.where(hit, jnp.float32(jnp.inf), s)


def _rank_sel_store(s, ref, slot):
    ii = jax.lax.broadcasted_iota(jnp.int32, (_NQ, _NQ), 1)
    cnt = jnp.zeros((_NQ, _NQ), jnp.int32)
    for i2 in range(_NQ):
        col = s[:, i2:i2 + 1]
        less = (col < s) | ((col == s) & (i2 < ii))
        cnt = cnt + less.astype(jnp.int32)
    for r in range(_K):
        ref[slot, r * _K:(r + 1) * _K, :] = jnp.where(cnt == r, 1.0, 0.0)


def _blockmax(x):
    z = jnp.maximum(x[:, :-8], x[:, 8:])
    z = jnp.maximum(z[:, :-4], z[:, 4:])
    z = jnp.maximum(z[:, :-2], z[:, 2:])
    z = jnp.maximum(z[:, :-1], z[:, 1:])
    r = jax.lax.broadcasted_iota(jnp.int32, (241, _NQ), 0)
    c = jax.lax.broadcasted_iota(jnp.int32, (241, _NQ), 1)
    epick = jnp.where(r == c * _K, 1.0, 0.0)
    return jnp.dot(z, epick, preferred_element_type=jnp.float32)


def _gn_lrelu(x, gamma, beta, gsize):
    inv_n = 1.0 / (gsize * 256.0)
    parts = []
    for g in range(x.shape[0] // gsize):
        xg = x[g * gsize:(g + 1) * gsize, :]
        mu = jnp.sum(xg) * inv_n
        var = jnp.sum(xg * xg) * inv_n - mu * mu
        y = (xg - mu) * jax.lax.rsqrt(var + 1e-5)
        y = y * gamma[g * gsize:(g + 1) * gsize, :] + beta[g * gsize:(g + 1) * gsize, :]
        parts.append(jnp.where(y >= 0, y, 0.2 * y))
    return jnp.concatenate(parts, axis=0)


def _dgcnn_body(cq_ref, cqt_ref, cgt_ref, f_ref, fq_ref, w1_ref,
                g1_ref, b1_ref, w2_ref, g2_ref, b2_ref, out_ref, sel_ref,
                sel2_ref):
    b = pl.program_id(0)
    wslot = jax.lax.rem(b, 2)
    rslot = 1 - wslot


    cq = cq_ref[0]
    cqt = cqt_ref[0]
    cgt = cgt_ref[0]
    cqb = cq.astype(jnp.bfloat16)
    mm = jnp.dot(cqb, cgt.astype(jnp.bfloat16),
                 preferred_element_type=jnp.float32)
    gram = jnp.dot(cqb, cqt.astype(jnp.bfloat16),
                   preferred_element_type=jnp.float32)
    cqsq = jnp.sum(cq * cq, axis=1, keepdims=True)
    cgsq = jnp.sum(cgt * cgt, axis=0, keepdims=True)
    s1 = (-2.0 * mm + cqsq) + cgsq
    nt_row = jnp.sum(cqt * cqt, axis=0, keepdims=True)
    s2 = (-2.0 * gram + cqsq) + nt_row

    pi = jax.lax.broadcasted_iota(jnp.int32, (_P, _NQ), 0)
    qi = jax.lax.broadcasted_iota(jnp.int32, (_P, _NQ), 1)
    eblk = jnp.where(pi // _K == qi, 1.0, 0.0)

    sel = sel_ref[rslot]
    sel2 = sel2_ref[rslot]
    fq = fq_ref[0]
    fgt = jax.lax.dot_general(sel, f_ref[0], _NT,
                              preferred_element_type=jnp.float32)
    fqrep = jax.lax.dot_general(eblk, fq, _NT,
                                preferred_element_type=jnp.float32)
    f1t = jnp.concatenate([fgt - fqrep, fqrep], axis=1).astype(jnp.bfloat16)
    h1 = jax.lax.dot_general(w1_ref[...], f1t, _NT,
                             preferred_element_type=jnp.float32)

    st = {}
    inv1 = 1.0 / (128.0 * 256.0)
    inv2 = 1.0 / (96.0 * 256.0)

    def gn1_sums(g):
        xg = h1[g * 128:(g + 1) * 128, :]
        st[f"s{g}"] = jnp.sum(xg) * inv1
        st[f"q{g}"] = jnp.sum(xg * xg) * inv1

    def gn1_scalars():
        for g in range(4):
            mu = st[f"s{g}"]
            var = st[f"q{g}"] - mu * mu
            st[f"mu{g}"] = mu
            st[f"inv{g}"] = jax.lax.rsqrt(var + 1e-5)

    def gn1_norm(g):
        xg = h1[g * 128:(g + 1) * 128, :]
        y = (xg - st[f"mu{g}"]) * st[f"inv{g}"]
        y = y * g1_ref[g * 128:(g + 1) * 128, :] + b1_ref[g * 128:(g + 1) * 128, :]
        st[f"y{g}"] = jnp.where(y >= 0, y, 0.2 * y)

    def hmax_pair(g):
        st[f"h{g}"] = _blockmax(st[f"y{g}"])
        st[f"h{g + 1}"] = _blockmax(st[f"y{g + 1}"])

    def stage2_a():
        h = jnp.concatenate([st["h0"], st["h1"], st["h2"], st["h3"]], axis=0)
        st["hsel"] = jax.lax.dot_general(sel2, h, _NT,
                                         preferred_element_type=jnp.float32)
        st["hrep"] = jax.lax.dot_general(eblk, h, _NT,
                                         preferred_element_type=jnp.float32)

    def stage2_b():
        f2t = jnp.concatenate([st["hsel"] - st["hrep"], st["hrep"]],
                              axis=1).astype(jnp.bfloat16)
        st["h2"] = jax.lax.dot_general(w2_ref[...], f2t, _NT,
                                       preferred_element_type=jnp.float32)

    def gn2_sums():
        for g in range(4):
            xg = st["h2"][g * 96:(g + 1) * 96, :]
            mu = jnp.sum(xg) * inv2
            st[f"n{g}"] = mu
            st[f"v{g}"] = jax.lax.rsqrt(jnp.sum(xg * xg) * inv2 - mu * mu + 1e-5)

    def gn2_norm(g0, g1):
        for g in (g0, g1):
            xg = st["h2"][g * 96:(g + 1) * 96, :]
            y = (xg - st[f"n{g}"]) * st[f"v{g}"]
            y = y * g2_ref[g * 96:(g + 1) * 96, :] + b2_ref[g * 96:(g + 1) * 96, :]
            st[f"o{g}"] = jnp.where(y >= 0, y, 0.2 * y)

    def finish():
        out_ref[0] = jnp.concatenate(
            [_blockmax(st[f"o{g}"]) for g in range(4)], axis=0)

    tail = [
        lambda: gn1_sums(0), lambda: gn1_sums(1),
        lambda: gn1_sums(2), lambda: gn1_sums(3),
        gn1_scalars,
        lambda: gn1_norm(0), lambda: gn1_norm(1),
        lambda: gn1_norm(2), lambda: gn1_norm(3),
        lambda: hmax_pair(0), lambda: hmax_pair(2),
        stage2_a, stage2_b,
        lambda: _rank_sel_store(s2, sel2_ref, wslot),
        gn2_sums,
        lambda: (gn2_norm(0, 1), gn2_norm(2, 3), finish()),
    ]

    sa = s1[:8, :]
    sb = s1[8:, :]
    iota = jax.lax.broadcasted_iota(jnp.int32, (8, _G), 1)
    for r in range(_K):
        sa = _topk_iter(sa, iota, _G, sel_ref, wslot, r, 0)
        sb = _topk_iter(sb, iota, _G, sel_ref, wslot, r, 8)
    for r in range(_K):
        tail[r]()


@jax.jit
def kernel(coor, f, coor_q, f_q, W1, g1, b1, W2, g2, b2):
    cgt = jnp.pad(jnp.transpose(coor, (0, 2, 1)), ((0, 0), (0, 5), (0, 0)))
    cq = jnp.pad(coor_q, ((0, 0), (0, 0), (0, 5)))
    cqt = jnp.pad(jnp.transpose(coor_q, (0, 2, 1)), ((0, 0), (0, 5), (0, 0)))
    w1b16 = W1.astype(jnp.bfloat16)
    w2b16 = W2.astype(jnp.bfloat16)
    g1c = g1.reshape(512, 1)
    b1c = b1.reshape(512, 1)
    g2c = g2.reshape(384, 1)
    b2c = b2.reshape(384, 1)

    def topk_idx(*shape):
        return pl.BlockSpec(
            shape, lambda b: (jnp.minimum(b, _B - 1),) + (0,) * (len(shape) - 1))

    def dense_idx(*shape):
        return pl.BlockSpec(
            shape, lambda b: (jnp.maximum(b - 1, 0),) + (0,) * (len(shape) - 1))

    bcast = lambda *shape: pl.BlockSpec(shape, lambda b: (0,) * len(shape))

    return pl.pallas_call(
        _dgcnn_body,
        grid=(_B + 1,),
        in_specs=[
            topk_idx(1, _NQ, 8),
            topk_idx(1, 8, _NQ),
            topk_idx(1, 8, _G),
            dense_idx(1, _CIN, _G),
            dense_idx(1, _CIN, _NQ),
            bcast(512, 768),
            bcast(512, 1),
            bcast(512, 1),
            bcast(384, 1024),
            bcast(384, 1),
            bcast(384, 1),
        ],
        out_specs=dense_idx(1, 384, _NQ),
        out_shape=jax.ShapeDtypeStruct((_B, 384, _NQ), jnp.float32),
        scratch_shapes=[
            pltpu.VMEM((2, _P, _G), jnp.float32),
            pltpu.VMEM((2, _P, _NQ), jnp.float32),
        ],
    )(cq, cqt, cgt, f, f_q, w1b16, g1c, b1c, w2b16, g2c, b2c)


# speedup vs baseline: 1.9253x; 1.1260x over previous
import jax
import jax.numpy as jnp
from jax.experimental import pallas as pl
from jax.experimental.pallas import tpu as pltpu

_K = 16
_B = 32
_SB = 2
_NB = _B // _SB
_G = 4096
_NQ = 16
_CIN = 384
_P = _K * _NQ
_NT = (((1,), (1,)), ((), ()))


def _rowmin(x):
    n = x.shape[1]
    while n > 128:
        n //= 2
        x = jnp.minimum(x[:, :n], x[:, n:])
    return jnp.min(x, axis=1, keepdims=True)


def _topk_iter(s, iota, ncand, store, r, row_off):
    m = _rowmin(s)
    t = jnp.where(s == m, iota, ncand)
    am = _rowmin(t)
    hit = t == am
    store(r, row_off, jnp.where(hit, 1.0, 0.0))
    return jnp.where(hit, jnp.float32(jnp.inf), s)


def _rank_ranks(s):
    ii = jax.lax.broadcasted_iota(jnp.int32, (_NQ, _NQ), 1)
    cnt = jnp.zeros((_NQ, _NQ), jnp.int32)
    for i2 in range(_NQ):
        col = s[:, i2:i2 + 1]
        less = (col < s) | ((col == s) & (i2 < ii))
        cnt = cnt + less.astype(jnp.int32)
    return cnt


def _blockmax(x):
    z = jnp.maximum(x[:, :-8], x[:, 8:])
    z = jnp.maximum(z[:, :-4], z[:, 4:])
    z = jnp.maximum(z[:, :-2], z[:, 2:])
    z = jnp.maximum(z[:, :-1], z[:, 1:])
    r = jax.lax.broadcasted_iota(jnp.int32, (241, _NQ), 0)
    c = jax.lax.broadcasted_iota(jnp.int32, (241, _NQ), 1)
    epick = jnp.where(r == c * _K, 1.0, 0.0)
    return jnp.dot(z, epick, preferred_element_type=jnp.float32)


def _dgcnn_body(cq_ref, cqt_ref, cgt_ref, f_ref, fq_ref, w1_ref,
                g1_ref, b1_ref, w2_ref, g2_ref, b2_ref, out_ref, sel_ref,
                sel2_ref):
    b = pl.program_id(0)
    wslot = jax.lax.rem(b, 2)
    rslot = 1 - wslot

    pi = jax.lax.broadcasted_iota(jnp.int32, (_P, _NQ), 0)
    qi = jax.lax.broadcasted_iota(jnp.int32, (_P, _NQ), 1)
    eblk = jnp.where(pi // _K == qi, 1.0, 0.0)

    s1s, s2s = [], []
    for sub in range(_SB):
        cq = cq_ref[sub]
        cqt = cqt_ref[sub]
        cgt = cgt_ref[sub]
        cqb = cq.astype(jnp.bfloat16)
        mm = jnp.dot(cqb, cgt.astype(jnp.bfloat16),
                     preferred_element_type=jnp.float32)
        gram = jnp.dot(cqb, cqt.astype(jnp.bfloat16),
                       preferred_element_type=jnp.float32)
        cqsq = jnp.sum(cq * cq, axis=1, keepdims=True)
        cgsq = jnp.sum(cgt * cgt, axis=0, keepdims=True)
        s1s.append((-2.0 * mm + cqsq) + cgsq)
        nt_row = jnp.sum(cqt * cqt, axis=0, keepdims=True)
        s2s.append((-2.0 * gram + cqsq) + nt_row)

    h1s, sel2s = [], []
    for sub in range(_SB):
        sel = sel_ref[rslot, sub]
        sel2s.append(sel2_ref[rslot, sub])
        fq = fq_ref[sub]
        fgt = jax.lax.dot_general(sel, f_ref[sub], _NT,
                                  preferred_element_type=jnp.float32)
        fqrep = jax.lax.dot_general(eblk, fq, _NT,
                                    preferred_element_type=jnp.float32)
        f1t = jnp.concatenate([fgt - fqrep, fqrep],
                              axis=1).astype(jnp.bfloat16)
        h1s.append(jax.lax.dot_general(w1_ref[...], f1t, _NT,
                                       preferred_element_type=jnp.float32))

    iota = jax.lax.broadcasted_iota(jnp.int32, (8, _G), 1)

    def mk_store(sub):
        def store(r, row_off, block):
            sel_ref[wslot, sub, r * _K + row_off:r * _K + row_off + 8, :] = block
        return store

    chains = []
    for sub in range(_SB):
        chains.append([s1s[sub][:8, :], s1s[sub][8:, :], mk_store(sub)])
    for r in range(_K):
        for sub in range(_SB):
            sa, sb_, store = chains[sub]
            chains[sub][0] = _topk_iter(sa, iota, _G, store, r, 0)
            chains[sub][1] = _topk_iter(sb_, iota, _G, store, r, 8)

    inv1 = 1.0 / (128.0 * 256.0)
    inv2 = 1.0 / (96.0 * 256.0)
    for sub in range(_SB):
        h1 = h1s[sub]
        ys = []
        for g in range(4):
            xg = h1[g * 128:(g + 1) * 128, :]
            mu = jnp.sum(xg) * inv1
            var = jnp.sum(xg * xg) * inv1 - mu * mu
            y = (xg - mu) * jax.lax.rsqrt(var + 1e-5)
            y = (y * g1_ref[g * 128:(g + 1) * 128, :]
                 + b1_ref[g * 128:(g + 1) * 128, :])
            ys.append(jnp.where(y >= 0, y, 0.2 * y))
        h = jnp.concatenate([_blockmax(y) for y in ys], axis=0)

        hsel = jax.lax.dot_general(sel2s[sub], h, _NT,
                                   preferred_element_type=jnp.float32)
        hrep = jax.lax.dot_general(eblk, h, _NT,
                                   preferred_element_type=jnp.float32)
        f2t = jnp.concatenate([hsel - hrep, hrep],
                              axis=1).astype(jnp.bfloat16)
        h2 = jax.lax.dot_general(w2_ref[...], f2t, _NT,
                                 preferred_element_type=jnp.float32)

        cnt = _rank_ranks(s2s[sub])
        for r in range(_K):
            sel2_ref[wslot, sub, r * _K:(r + 1) * _K, :] = (
                jnp.where(cnt == r, 1.0, 0.0))

        os_ = []
        for g in range(4):
            xg = h2[g * 96:(g + 1) * 96, :]
            mu = jnp.sum(xg) * inv2
            inv = jax.lax.rsqrt(jnp.sum(xg * xg) * inv2 - mu * mu + 1e-5)
            y = (xg - mu) * inv
            y = (y * g2_ref[g * 96:(g + 1) * 96, :]
                 + b2_ref[g * 96:(g + 1) * 96, :])
            os_.append(jnp.where(y >= 0, y, 0.2 * y))
        out_ref[sub] = jnp.concatenate(
            [_blockmax(o) for o in os_], axis=0)


@jax.jit
def kernel(coor, f, coor_q, f_q, W1, g1, b1, W2, g2, b2):
    cgt = jnp.pad(jnp.transpose(coor, (0, 2, 1)), ((0, 0), (0, 5), (0, 0)))
    cq = jnp.pad(coor_q, ((0, 0), (0, 0), (0, 5)))
    cqt = jnp.pad(jnp.transpose(coor_q, (0, 2, 1)), ((0, 0), (0, 5), (0, 0)))
    w1b16 = W1.astype(jnp.bfloat16)
    w2b16 = W2.astype(jnp.bfloat16)
    g1c = g1.reshape(512, 1)
    b1c = b1.reshape(512, 1)
    g2c = g2.reshape(384, 1)
    b2c = b2.reshape(384, 1)

    def topk_idx(*shape):
        return pl.BlockSpec(
            shape, lambda b: (jnp.minimum(b, _NB - 1),) + (0,) * (len(shape) - 1))

    def dense_idx(*shape):
        return pl.BlockSpec(
            shape, lambda b: (jnp.maximum(b - 1, 0),) + (0,) * (len(shape) - 1))

    bcast = lambda *shape: pl.BlockSpec(shape, lambda b: (0,) * len(shape))

    return pl.pallas_call(
        _dgcnn_body,
        grid=(_NB + 1,),
        in_specs=[
            topk_idx(_SB, _NQ, 8),
            topk_idx(_SB, 8, _NQ),
            topk_idx(_SB, 8, _G),
            dense_idx(_SB, _CIN, _G),
            dense_idx(_SB, _CIN, _NQ),
            bcast(512, 768),
            bcast(512, 1),
            bcast(512, 1),
            bcast(384, 1024),
            bcast(384, 1),
            bcast(384, 1),
        ],
        out_specs=dense_idx(_SB, 384, _NQ),
        out_shape=jax.ShapeDtypeStruct((_B, 384, _NQ), jnp.float32),
        scratch_shapes=[
            pltpu.VMEM((2, _SB, _P, _G), jnp.float32),
            pltpu.VMEM((2, _SB, _P, _NQ), jnp.float32),
        ],
    )(cq, cqt, cgt, f, f_q, w1b16, g1c, b1c, w2b16, g2c, b2c)
